# baseline (speedup 1.0000x reference)
import functools
import jax
import jax.numpy as jnp
import numpy as np
from jax import lax
from jax.experimental import pallas as pl
from jax.experimental.pallas import tpu as pltpu

N = 100000
E = 1600000
G = 128
H = 32
K = 30
TOTAL_LATENT = H * 3 + 1
DENSE_DIM = ((K - 2) // 2 + 1 - 5 + 1) * 32


def _head_body(p_ref, w1_ref, b1_ref, w2_ref, b2_ref, l1_ref, l1b_ref,
               l2_ref, l2b_ref, out_ref):
    p = p_ref[...]
    w1 = w1_ref[...]
    b1 = b1_ref[...]
    y1 = []
    for k in range(K):
        seg = p[:, k * TOTAL_LATENT:(k + 1) * TOTAL_LATENT]
        y1.append(jax.nn.relu(
            jnp.dot(seg, w1, preferred_element_type=jnp.float32) + b1))
    pooled = [jnp.maximum(y1[2 * t], y1[2 * t + 1]) for t in range(K // 2)]
    w2 = w2_ref[...]
    b2 = b2_ref[...]
    dcols = []
    for t in range(11):
        patch = jnp.concatenate(pooled[t:t + 5], axis=1)
        dcols.append(jax.nn.relu(
            jnp.dot(patch, w2, preferred_element_type=jnp.float32) + b2))
    d = jnp.concatenate(dcols, axis=1)
    h1 = jax.nn.relu(
        jnp.dot(d, l1_ref[...], preferred_element_type=jnp.float32)
        + l1b_ref[...])
    out_ref[...] = (jnp.dot(h1, l2_ref[...],
                            preferred_element_type=jnp.float32)
                    + l2b_ref[...])


def _head(p, conv1_w, conv1_b, conv2_w, conv2_b, lin1_w, lin1_b,
          lin2_w, lin2_b):
    w1 = conv1_w[:, 0, :].T
    w2 = jnp.transpose(conv2_w, (2, 1, 0)).reshape(80, 32)
    perm = np.zeros((DENSE_DIM,), np.int32)
    for t in range(11):
        for o2 in range(32):
            perm[t * 32 + o2] = o2 * 11 + t
    l1 = lin1_w[:, perm].T
    l2 = lin2_w.T
    return pl.pallas_call(
        _head_body,
        out_shape=jax.ShapeDtypeStruct((G, 1), jnp.float32),
    )(p, w1, conv1_b[None, :], w2, conv2_b[None, :],
      l1, lin1_b[None, :], l2, lin2_b[None, :])


def _gcn_layer(x, src, dst, W, b, n):
    x = x @ W
    loop = jnp.arange(n, dtype=src.dtype)
    s = jnp.concatenate([src, loop])
    d = jnp.concatenate([dst, loop])
    deg = jnp.zeros((n,), x.dtype).at[d].add(1.0)
    dinv = jnp.where(deg > 0, deg ** -0.5, 0.0)
    norm = dinv[s] * dinv[d]
    out = jnp.zeros_like(x).at[d].add(x[s] * norm[:, None])
    return out + b


def _pool(x, batch, k, g):
    order = jnp.lexsort((-x[:, -1], batch))
    xs = x[order]
    bs = batch[order]
    counts = jnp.zeros((g,), jnp.int32).at[bs].add(1)
    starts = jnp.cumsum(counts) - counts
    pos = jnp.arange(x.shape[0], dtype=jnp.int32) - starts[bs]
    out = jnp.zeros((g, k, x.shape[1]), x.dtype)
    out = out.at[bs, pos].set(xs, mode='drop')
    return out.reshape(g, k * x.shape[1])


def kernel(z, edge_index, batch, use_feature, embedding, z_table,
           W0, b0, W1, b1, W2, b2, W3, b3,
           conv1_w, conv1_b, conv2_w, conv2_b,
           lin1_w, lin1_b, lin2_w, lin2_b):
    n = z.shape[0]
    x = z_table[z]
    src, dst = edge_index[0], edge_index[1]
    x1 = jnp.tanh(_gcn_layer(x, src, dst, W0, b0, n))
    x2 = jnp.tanh(_gcn_layer(x1, src, dst, W1, b1, n))
    x3 = jnp.tanh(_gcn_layer(x2, src, dst, W2, b2, n))
    x4 = jnp.tanh(_gcn_layer(x3, src, dst, W3, b3, n))
    xcat = jnp.concatenate([x1, x2, x3, x4], axis=-1)
    p = _pool(xcat, batch, K, G)
    return _head(p, conv1_w, conv1_b, conv2_w, conv2_b,
                 lin1_w, lin1_b, lin2_w, lin2_b)


# speedup vs baseline: 1.0001x; 1.0001x over previous
import jax
import jax.numpy as jnp
import numpy as np
from jax.experimental import pallas as pl
from jax.experimental.pallas import tpu as pltpu

N = 100000
E = 1600000
G = 128
H = 32
K = 30
TOTAL_LATENT = H * 3 + 1
DENSE_DIM = ((K - 2) // 2 + 1 - 5 + 1) * 32


def _head_body(p_ref, w1_ref, b1_ref, w2_ref, b2_ref, l1_ref, l1b_ref,
               l2_ref, l2b_ref, out_ref):
    p = p_ref[...]
    w1 = w1_ref[...]
    b1 = b1_ref[...]
    y1 = []
    for k in range(K):
        seg = p[:, k * TOTAL_LATENT:(k + 1) * TOTAL_LATENT]
        y1.append(jax.nn.relu(
            jnp.dot(seg, w1, preferred_element_type=jnp.float32) + b1))
    pooled = [jnp.maximum(y1[2 * t], y1[2 * t + 1]) for t in range(K // 2)]
    w2 = w2_ref[...]
    b2 = b2_ref[...]
    dcols = []
    for t in range(11):
        patch = jnp.concatenate(pooled[t:t + 5], axis=1)
        dcols.append(jax.nn.relu(
            jnp.dot(patch, w2, preferred_element_type=jnp.float32) + b2))
    d = jnp.concatenate(dcols, axis=1)
    h1 = jax.nn.relu(
        jnp.dot(d, l1_ref[...], preferred_element_type=jnp.float32)
        + l1b_ref[...])
    out_ref[...] = (jnp.dot(h1, l2_ref[...],
                            preferred_element_type=jnp.float32)
                    + l2b_ref[...])


def _head(p, conv1_w, conv1_b, conv2_w, conv2_b, lin1_w, lin1_b,
          lin2_w, lin2_b):
    w1 = conv1_w[:, 0, :].T
    w2 = jnp.transpose(conv2_w, (2, 1, 0)).reshape(80, 32)
    perm = np.zeros((DENSE_DIM,), np.int32)
    for t in range(11):
        for o2 in range(32):
            perm[t * 32 + o2] = o2 * 11 + t
    l1 = lin1_w[:, perm].T
    l2 = lin2_w.T
    return pl.pallas_call(
        _head_body,
        out_shape=jax.ShapeDtypeStruct((G, 1), jnp.float32),
    )(p, w1, conv1_b[None, :], w2, conv2_b[None, :],
      l1, lin1_b[None, :], l2, lin2_b[None, :])


def _pool(x, batch, k, g):
    order = jnp.lexsort((-x[:, -1], batch))
    xs = x[order]
    bs = batch[order]
    counts = jnp.zeros((g,), jnp.int32).at[bs].add(1)
    starts = jnp.cumsum(counts) - counts
    pos = jnp.arange(x.shape[0], dtype=jnp.int32) - starts[bs]
    out = jnp.zeros((g, k, x.shape[1]), x.dtype)
    out = out.at[bs, pos].set(xs, mode='drop')
    return out.reshape(g, k * x.shape[1])


def _gcn_layer(x, src, dst, W, b, n):
    x = x @ W
    loop = jnp.arange(n, dtype=src.dtype)
    s = jnp.concatenate([src, loop])
    d = jnp.concatenate([dst, loop])
    deg = jnp.zeros((n,), x.dtype).at[d].add(1.0)
    dinv = jnp.where(deg > 0, deg ** -0.5, 0.0)
    norm = dinv[s] * dinv[d]
    out = jnp.zeros_like(x).at[d].add(x[s] * norm[:, None])
    return out + b


def kernel(z, edge_index, batch, use_feature, embedding, z_table,
           W0, b0, W1, b1, W2, b2, W3, b3,
           conv1_w, conv1_b, conv2_w, conv2_b,
           lin1_w, lin1_b, lin2_w, lin2_b):
    src, dst = edge_index[0], edge_index[1]
    x = z_table[z]
    x1 = jnp.tanh(_gcn_layer(x, src, dst, W0, b0, N))
    x2 = jnp.tanh(_gcn_layer(x1, src, dst, W1, b1, N))
    x3 = jnp.tanh(_gcn_layer(x2, src, dst, W2, b2, N))
    x4 = jnp.tanh(_gcn_layer(x3, src, dst, W3, b3, N))
    xcat = jnp.concatenate([x1, x2, x3, x4], axis=-1)
    p = _pool(xcat, batch, K, G)
    return _head(p, conv1_w, conv1_b, conv2_w, conv2_b,
                 lin1_w, lin1_b, lin2_w, lin2_b)
